# baseline (speedup 1.0000x reference)
import functools

import jax
import jax.numpy as jnp
from jax import lax
from jax.experimental import pallas as pl
from jax.experimental.pallas import tpu as pltpu
from jax.experimental.pallas import tpu_sc as plsc

N = 10000
E = 160000
IN, HID, OUT = 256, 512, 256
ALPHA = 0.5
H = 128
NT = 16
EPT = E // NT
CH = 80
CPT = CH * 128
PAD = CPT - EPT
ACC_R = N + 16

_mesh = plsc.VectorSubcoreMesh(core_axis_name="c", subcore_axis_name="s")


def _make_segsum(with_counts):
    if with_counts:
        out_type = (
            jax.ShapeDtypeStruct((2 * N, H), jnp.float32),
            jax.ShapeDtypeStruct((2 * N, H), jnp.float32),
            jax.ShapeDtypeStruct((N,), jnp.float32),
            jax.ShapeDtypeStruct((N,), jnp.float32),
        )
    else:
        out_type = (
            jax.ShapeDtypeStruct((2 * N, H), jnp.float32),
            jax.ShapeDtypeStruct((2 * N, H), jnp.float32),
        )

    @functools.partial(
        pl.kernel,
        out_type=out_type,
        mesh=_mesh,
        scratch_types=[
            pltpu.VMEM((CH // 2, 128), jnp.int32),
            pltpu.VMEM((CH // 2, 128), jnp.int32),
            pltpu.VMEM((128, H), jnp.float32),
            pltpu.VMEM((128, H), jnp.float32),
            pltpu.VMEM((128,), jnp.float32),
            pltpu.VMEM((640,), jnp.float32),
            pltpu.VMEM_SHARED((ACC_R, H), jnp.float32),
            pltpu.VMEM_SHARED((ACC_R,), jnp.float32),
            pltpu.SemaphoreType.DMA,
            pltpu.SemaphoreType.DMA,
        ],
    )
    def _segsum(table_in, table_out, src_g, src_g1, dst_g, dst_g1,
                src_s, dst_s, *rest):
        if with_counts:
            (out_in, out_out, out_cin, out_cout, idxg, idxs, rows_a, rows_b,
             ones, zflat, acc, cnt, sem_a, sem_b) = rest
        else:
            (out_in, out_out, idxg, idxs, rows_a, rows_b,
             ones, zflat, acc, cnt, sem_a, sem_b) = rest
            out_cin = out_cout = None
        c = lax.axis_index("c")
        s = lax.axis_index("s")
        HC = CH // 2

        zv = jnp.zeros((16,), jnp.float32)
        ov = jnp.ones((16,), jnp.float32)
        base = s * 632
        db = s * 632

        if with_counts:
            for k in range(128 // 16):
                ones[pl.ds(k * 16, 16)] = ov

        def _zero_rows_a():
            def _zr(i, carry):
                for k in range(H // 16):
                    rows_a[i, pl.ds(k * 16, 16)] = zv
                return carry

            lax.fori_loop(0, 128, _zr, 0)

        def _zero_zflat():
            for k in range(640 // 16):
                zflat[pl.ds(k * 16, 16)] = zv

        def _zero_acc():
            for j in range(4):
                pltpu.sync_copy(rows_a, acc.at[pl.ds(base + j * 128, 128)])

            @pl.when(s < 15)
            def _():
                pltpu.sync_copy(rows_a.at[pl.ds(0, 120)],
                                acc.at[pl.ds(base + 512, 120)])

            @pl.when(s == 15)
            def _():
                pltpu.sync_copy(rows_a.at[pl.ds(0, 24)],
                                acc.at[pl.ds(base + 512, 24)])

        def _zero_cnt():
            @pl.when(s < 15)
            def _():
                pltpu.sync_copy(zflat, cnt.at[pl.ds(s * 640, 640)])

            @pl.when(s == 15)
            def _():
                pltpu.sync_copy(zflat.at[pl.ds(0, 416)],
                                cnt.at[pl.ds(9600, 416)])

        def _run_phase(table, g_lo, g_hi, s_idx, out_sum, out_cnt):
            def _gather(ch, buf, sem):
                return pltpu.async_copy(table.at[idxg.at[ch]], buf, sem)

            def _drain(ch, buf, sem):
                pltpu.make_async_copy(table.at[idxg.at[ch]], buf, sem).wait()

            def _scatter(ch, buf):
                pltpu.sync_copy(buf, acc.at[idxs.at[ch]], add=True)
                if out_cnt is not None:
                    pltpu.sync_copy(ones, cnt.at[idxs.at[ch]], add=True)

            for half in range(2):
                pltpu.sync_copy(s_idx.at[s, pl.ds(half * HC, HC)], idxs)

                @pl.when(c == 0)
                def _():
                    pltpu.sync_copy(g_lo.at[s, pl.ds(half * HC, HC)], idxg)

                @pl.when(c == 1)
                def _():
                    pltpu.sync_copy(g_hi.at[s, pl.ds(half * HC, HC)], idxg)

                _gather(0, rows_a, sem_a)

                def _body(j, carry):
                    a = 2 * j
                    b = a + 1
                    _gather(b, rows_b, sem_b)
                    _drain(a, rows_a, sem_a)
                    _scatter(a, rows_a)

                    @pl.when(j < HC // 2 - 1)
                    def _():
                        _gather(a + 2, rows_a, sem_a)

                    _drain(b, rows_b, sem_b)
                    _scatter(b, rows_b)
                    return carry

                lax.fori_loop(0, HC // 2, _body, 0)

            plsc.subcore_barrier()

            @pl.when(s < 15)
            def _():
                pltpu.sync_copy(acc.at[pl.ds(db, 632)],
                                out_sum.at[pl.ds(c * N + db, 632)])

            @pl.when(s == 15)
            def _():
                pltpu.sync_copy(acc.at[pl.ds(9480, 520)],
                                out_sum.at[pl.ds(c * N + 9480, 520)])

            if out_cnt is not None:
                @pl.when(c == 0)
                def _():
                    @pl.when(s < 15)
                    def _():
                        pltpu.sync_copy(cnt.at[pl.ds(s * 640, 640)], zflat)
                        pltpu.sync_copy(zflat, out_cnt.at[pl.ds(s * 640, 640)])

                    @pl.when(s == 15)
                    def _():
                        pltpu.sync_copy(cnt.at[pl.ds(9600, 400)],
                                        zflat.at[pl.ds(0, 400)])
                        pltpu.sync_copy(zflat.at[pl.ds(0, 400)],
                                        out_cnt.at[pl.ds(9600, 400)])

        _zero_rows_a()
        if with_counts:
            _zero_zflat()
            _zero_cnt()
        _zero_acc()
        plsc.subcore_barrier()
        _run_phase(table_in, src_g, src_g1, dst_s, out_in, out_cin)

        _zero_rows_a()
        if with_counts:
            _zero_zflat()
            _zero_cnt()
        _zero_acc()
        plsc.subcore_barrier()
        _run_phase(table_out, dst_g, dst_g1, src_s, out_out, out_cout)

    return _segsum


_segsum_cnt = _make_segsum(True)
_segsum_nocnt = _make_segsum(False)


M1 = 1000


def _tc12_body(x, slo, shi, ulo, uhi, cin, cout, ws, wi, wo, b, wcat,
               s2, p, q):
    rin = 1.0 / jnp.maximum(cin[...], 1.0)
    rout = 1.0 / jnp.maximum(cout[...], 1.0)
    tin = (jnp.dot(slo[...], wi[0:H, :], preferred_element_type=jnp.float32)
           + jnp.dot(shi[...], wi[H:IN, :], preferred_element_type=jnp.float32))
    tout = (jnp.dot(ulo[...], wo[0:H, :], preferred_element_type=jnp.float32)
            + jnp.dot(uhi[...], wo[H:IN, :], preferred_element_type=jnp.float32))
    hs = jnp.dot(x[...], ws[...], preferred_element_type=jnp.float32)
    h = jnp.maximum(hs + tin * rin + tout * rout + b[...], 0.0)
    g = jnp.dot(h, wcat[...], preferred_element_type=jnp.float32)
    s2[...] = g[:, 0:OUT]
    p[...] = g[:, OUT:2 * OUT]
    q[...] = g[:, 2 * OUT:3 * OUT]


def _tc3_body(s2, slo, shi, ulo, uhi, cin, cout, b, out):
    rin = 1.0 / jnp.maximum(cin[...], 1.0)
    rout = 1.0 / jnp.maximum(cout[...], 1.0)
    lo = slo[...] * rin + ulo[...] * rout
    hi = shi[...] * rin + uhi[...] * rout
    out[...] = s2[...] + jnp.concatenate([lo, hi], axis=1) + b[...]


def _row_spec(w):
    return pl.BlockSpec((M1, w), lambda i: (i, 0))


def _full_spec(r, w):
    return pl.BlockSpec((r, w), lambda i: (0, 0))


def kernel(x, edge_index, W_in1, b_in1, W_out1, b_out1, W_self1, b_self1,
           W_in2, b_in2, W_out2, b_out2, W_self2, b_self2):
    f32 = jnp.float32
    src = edge_index[0]
    dst = edge_index[1]

    spread = jnp.arange(PAD, dtype=jnp.int32) % 16

    def mk(v, padv):
        t = jnp.concatenate(
            [v.reshape(NT, EPT),
             jnp.broadcast_to(padv[None, :], (NT, PAD))], axis=1)
        return t.reshape(NT, CH, 128)

    src_g = mk(2 * src, 2 * spread)
    dst_g = mk(2 * dst, 2 * spread)
    src_s = mk(src, N + spread)
    dst_s = mk(dst, N + spread)
    src_g1 = src_g + 1
    dst_g1 = dst_g + 1

    x2 = x.reshape(2 * N, H)

    s1, u1, cin, cout = _segsum_cnt(x2, x2, src_g, src_g1, dst_g, dst_g1,
                                    src_s, dst_s)

    cin2 = cin.reshape(N, 1)
    cout2 = cout.reshape(N, 1)

    wi1 = (1.0 - ALPHA) * W_in1
    wo1 = ALPHA * W_out1
    b1 = (b_self1 + (1.0 - ALPHA) * b_in1 + ALPHA * b_out1).reshape(1, HID)
    wcat = jnp.concatenate(
        [W_self2, (1.0 - ALPHA) * W_in2, ALPHA * W_out2], axis=1)

    s2, p, q = pl.pallas_call(
        _tc12_body,
        grid=(N // M1,),
        in_specs=[
            _row_spec(IN), _row_spec(H), _row_spec(H), _row_spec(H),
            _row_spec(H), _row_spec(1), _row_spec(1),
            _full_spec(IN, HID), _full_spec(IN, HID), _full_spec(IN, HID),
            _full_spec(1, HID), _full_spec(HID, 3 * OUT),
        ],
        out_specs=[_row_spec(OUT), _row_spec(OUT), _row_spec(OUT)],
        out_shape=[
            jax.ShapeDtypeStruct((N, OUT), f32),
            jax.ShapeDtypeStruct((N, OUT), f32),
            jax.ShapeDtypeStruct((N, OUT), f32),
        ],
    )(x, s1[0:N], s1[N:2 * N], u1[0:N], u1[N:2 * N], cin2, cout2,
      W_self1, wi1, wo1, b1, wcat)

    s2sum, u2sum = _segsum_nocnt(p.reshape(2 * N, H), q.reshape(2 * N, H),
                                 src_g, src_g1, dst_g, dst_g1, src_s, dst_s)

    b2 = (b_self2 + (1.0 - ALPHA) * b_in2 + ALPHA * b_out2).reshape(1, OUT)
    out = pl.pallas_call(
        _tc3_body,
        grid=(N // M1,),
        in_specs=[
            _row_spec(OUT), _row_spec(H), _row_spec(H), _row_spec(H),
            _row_spec(H), _row_spec(1), _row_spec(1), _full_spec(1, OUT),
        ],
        out_specs=_row_spec(OUT),
        out_shape=jax.ShapeDtypeStruct((N, OUT), f32),
    )(s2, s2sum[0:N], s2sum[N:2 * N], u2sum[0:N], u2sum[N:2 * N],
      cin2, cout2, b2)
    return out


# speedup vs baseline: 1.0342x; 1.0342x over previous
import functools

import jax
import jax.numpy as jnp
from jax import lax
from jax.experimental import pallas as pl
from jax.experimental.pallas import tpu as pltpu
from jax.experimental.pallas import tpu_sc as plsc

N = 10000
E = 160000
IN, HID, OUT = 256, 512, 256
ALPHA = 0.5
H = 128
NT = 16
EPT = E // NT
CW = 64
NCH = 160
PAD = NCH * CW - EPT
ACC_R = N + 16

_mesh = plsc.VectorSubcoreMesh(core_axis_name="c", subcore_axis_name="s")


def _make_segsum(with_counts):
    if with_counts:
        out_type = (
            jax.ShapeDtypeStruct((2 * N, H), jnp.float32),
            jax.ShapeDtypeStruct((2 * N, H), jnp.float32),
            jax.ShapeDtypeStruct((N,), jnp.float32),
            jax.ShapeDtypeStruct((N,), jnp.float32),
        )
    else:
        out_type = (
            jax.ShapeDtypeStruct((2 * N, H), jnp.float32),
            jax.ShapeDtypeStruct((2 * N, H), jnp.float32),
        )

    @functools.partial(
        pl.kernel,
        out_type=out_type,
        mesh=_mesh,
        scratch_types=[
            pltpu.VMEM((NCH // 4, CW), jnp.int32),
            pltpu.VMEM((NCH // 4, CW), jnp.int32),
            pltpu.VMEM((CW, H), jnp.float32),
            pltpu.VMEM((CW, H), jnp.float32),
            pltpu.VMEM((CW, H), jnp.float32),
            pltpu.VMEM((CW, H), jnp.float32),
            pltpu.VMEM((CW,), jnp.float32),
            pltpu.VMEM((640,), jnp.float32),
            pltpu.VMEM_SHARED((ACC_R, H), jnp.float32),
            pltpu.VMEM_SHARED((ACC_R,), jnp.float32),
            pltpu.SemaphoreType.DMA,
            pltpu.SemaphoreType.DMA,
            pltpu.SemaphoreType.DMA,
        ],
    )
    def _segsum(table_in, table_out, src_g, src_g1, dst_g, dst_g1,
                src_s, dst_s, *rest):
        if with_counts:
            (out_in, out_out, out_cin, out_cout, idxg, idxs, r0, r1, r2, r3,
             ones, zflat, acc, cnt, sem_g, sem_s, sem_o) = rest
        else:
            (out_in, out_out, idxg, idxs, r0, r1, r2, r3,
             ones, zflat, acc, cnt, sem_g, sem_s, sem_o) = rest
            out_cin = out_cout = None
        c = lax.axis_index("c")
        s = lax.axis_index("s")
        bufs = (r0, r1, r2, r3)
        HC = NCH // 4

        zv = jnp.zeros((16,), jnp.float32)
        ov = jnp.ones((16,), jnp.float32)
        base = s * 632
        db = s * 632

        if with_counts:
            for k in range(CW // 16):
                ones[pl.ds(k * 16, 16)] = ov

        def _zero_rows_a():
            def _zr(i, carry):
                for k in range(H // 16):
                    r0[i, pl.ds(k * 16, 16)] = zv
                return carry

            lax.fori_loop(0, CW, _zr, 0)

        def _zero_zflat():
            for k in range(640 // 16):
                zflat[pl.ds(k * 16, 16)] = zv

        def _zero_acc():
            for j in range(8):
                pltpu.sync_copy(r0, acc.at[pl.ds(base + j * CW, CW)])

            @pl.when(s < 15)
            def _():
                pltpu.sync_copy(r0, acc.at[pl.ds(base + 512, CW)])
                pltpu.sync_copy(r0.at[pl.ds(0, 56)],
                                acc.at[pl.ds(base + 576, 56)])

            @pl.when(s == 15)
            def _():
                pltpu.sync_copy(r0.at[pl.ds(0, 24)],
                                acc.at[pl.ds(base + 512, 24)])

        def _zero_cnt():
            @pl.when(s < 15)
            def _():
                pltpu.sync_copy(zflat, cnt.at[pl.ds(s * 640, 640)])

            @pl.when(s == 15)
            def _():
                pltpu.sync_copy(zflat.at[pl.ds(0, 416)],
                                cnt.at[pl.ds(9600, 416)])

        def _run_phase(table, g_lo, g_hi, s_idx, out_sum, out_cnt):
            def _gather(ch, buf):
                pltpu.async_copy(table.at[idxg.at[ch]], buf, sem_g)

            def _drain_g(ch, buf):
                pltpu.make_async_copy(table.at[idxg.at[ch]], buf,
                                      sem_g).wait()

            def _scatter(ch, buf):
                pltpu.async_copy(buf, acc.at[idxs.at[ch]], sem_s, add=True)
                if out_cnt is not None:
                    pltpu.async_copy(ones, cnt.at[idxs.at[ch]], sem_o,
                                     add=True)

            def _drain_s(ch, buf):
                pltpu.make_async_copy(buf, acc.at[idxs.at[ch]],
                                      sem_s).wait()

            for half in range(4):
                pltpu.sync_copy(s_idx.at[s, pl.ds(half * HC, HC)], idxs)

                @pl.when(c == 0)
                def _():
                    pltpu.sync_copy(g_lo.at[s, pl.ds(half * HC, HC)], idxg)

                @pl.when(c == 1)
                def _():
                    pltpu.sync_copy(g_hi.at[s, pl.ds(half * HC, HC)], idxg)

                for ch in range(3):
                    _gather(ch, bufs[ch])

                def _body(jj, carry):
                    for b in range(4):
                        ch = 4 * jj + b
                        _drain_g(ch, bufs[b])
                        _scatter(ch, bufs[b])
                        if b == 0:
                            @pl.when(jj > 0)
                            def _():
                                _drain_s(ch - 1, bufs[3])
                            _gather(ch + 3, bufs[3])
                        else:
                            _drain_s(ch - 1, bufs[b - 1])

                            @pl.when(jj < HC // 4 - 1)
                            def _():
                                _gather(ch + 3, bufs[b - 1])
                    return carry

                lax.fori_loop(0, HC // 4, _body, 0)
                _drain_s(HC - 1, bufs[3])
                if out_cnt is not None:
                    def _dro(ch, carry):
                        pltpu.make_async_copy(ones, cnt.at[idxs.at[ch]],
                                              sem_o).wait()
                        return carry

                    lax.fori_loop(0, HC, _dro, 0)

            plsc.subcore_barrier()

            @pl.when(s < 15)
            def _():
                pltpu.sync_copy(acc.at[pl.ds(db, 632)],
                                out_sum.at[pl.ds(c * N + db, 632)])

            @pl.when(s == 15)
            def _():
                pltpu.sync_copy(acc.at[pl.ds(9480, 520)],
                                out_sum.at[pl.ds(c * N + 9480, 520)])

            if out_cnt is not None:
                @pl.when(c == 0)
                def _():
                    @pl.when(s < 15)
                    def _():
                        pltpu.sync_copy(cnt.at[pl.ds(s * 640, 640)], zflat)
                        pltpu.sync_copy(zflat, out_cnt.at[pl.ds(s * 640, 640)])

                    @pl.when(s == 15)
                    def _():
                        pltpu.sync_copy(cnt.at[pl.ds(9600, 400)],
                                        zflat.at[pl.ds(0, 400)])
                        pltpu.sync_copy(zflat.at[pl.ds(0, 400)],
                                        out_cnt.at[pl.ds(9600, 400)])

        _zero_rows_a()
        if with_counts:
            _zero_zflat()
            _zero_cnt()
        _zero_acc()
        plsc.subcore_barrier()
        _run_phase(table_in, src_g, src_g1, dst_s, out_in, out_cin)

        _zero_rows_a()
        if with_counts:
            _zero_zflat()
            _zero_cnt()
        _zero_acc()
        plsc.subcore_barrier()
        _run_phase(table_out, dst_g, dst_g1, src_s, out_out, out_cout)

    return _segsum


_segsum_cnt = _make_segsum(True)
_segsum_nocnt = _make_segsum(False)


M1 = 1000


def _tc12_body(x, slo, shi, ulo, uhi, cin, cout, ws, wi, wo, b, wcat,
               s2, p, q):
    rin = 1.0 / jnp.maximum(cin[...], 1.0)
    rout = 1.0 / jnp.maximum(cout[...], 1.0)
    tin = (jnp.dot(slo[...], wi[0:H, :], preferred_element_type=jnp.float32)
           + jnp.dot(shi[...], wi[H:IN, :], preferred_element_type=jnp.float32))
    tout = (jnp.dot(ulo[...], wo[0:H, :], preferred_element_type=jnp.float32)
            + jnp.dot(uhi[...], wo[H:IN, :], preferred_element_type=jnp.float32))
    hs = jnp.dot(x[...], ws[...], preferred_element_type=jnp.float32)
    h = jnp.maximum(hs + tin * rin + tout * rout + b[...], 0.0)
    g = jnp.dot(h, wcat[...], preferred_element_type=jnp.float32)
    s2[...] = g[:, 0:OUT]
    p[...] = g[:, OUT:2 * OUT]
    q[...] = g[:, 2 * OUT:3 * OUT]


def _tc3_body(s2, slo, shi, ulo, uhi, cin, cout, b, out):
    rin = 1.0 / jnp.maximum(cin[...], 1.0)
    rout = 1.0 / jnp.maximum(cout[...], 1.0)
    lo = slo[...] * rin + ulo[...] * rout
    hi = shi[...] * rin + uhi[...] * rout
    out[...] = s2[...] + jnp.concatenate([lo, hi], axis=1) + b[...]


def _row_spec(w):
    return pl.BlockSpec((M1, w), lambda i: (i, 0))


def _full_spec(r, w):
    return pl.BlockSpec((r, w), lambda i: (0, 0))


def kernel(x, edge_index, W_in1, b_in1, W_out1, b_out1, W_self1, b_self1,
           W_in2, b_in2, W_out2, b_out2, W_self2, b_self2):
    f32 = jnp.float32
    src = edge_index[0]
    dst = edge_index[1]

    spread = jnp.arange(PAD, dtype=jnp.int32) % 16

    def mk(v, padv):
        t = jnp.concatenate(
            [v.reshape(NT, EPT),
             jnp.broadcast_to(padv[None, :], (NT, PAD))], axis=1)
        return t.reshape(NT, NCH, CW)

    src_g = mk(2 * src, 2 * spread)
    dst_g = mk(2 * dst, 2 * spread)
    src_s = mk(src, N + spread)
    dst_s = mk(dst, N + spread)
    src_g1 = src_g + 1
    dst_g1 = dst_g + 1

    x2 = x.reshape(2 * N, H)

    s1, u1, cin, cout = _segsum_cnt(x2, x2, src_g, src_g1, dst_g, dst_g1,
                                    src_s, dst_s)

    cin2 = cin.reshape(N, 1)
    cout2 = cout.reshape(N, 1)

    wi1 = (1.0 - ALPHA) * W_in1
    wo1 = ALPHA * W_out1
    b1 = (b_self1 + (1.0 - ALPHA) * b_in1 + ALPHA * b_out1).reshape(1, HID)
    wcat = jnp.concatenate(
        [W_self2, (1.0 - ALPHA) * W_in2, ALPHA * W_out2], axis=1)

    s2, p, q = pl.pallas_call(
        _tc12_body,
        grid=(N // M1,),
        in_specs=[
            _row_spec(IN), _row_spec(H), _row_spec(H), _row_spec(H),
            _row_spec(H), _row_spec(1), _row_spec(1),
            _full_spec(IN, HID), _full_spec(IN, HID), _full_spec(IN, HID),
            _full_spec(1, HID), _full_spec(HID, 3 * OUT),
        ],
        out_specs=[_row_spec(OUT), _row_spec(OUT), _row_spec(OUT)],
        out_shape=[
            jax.ShapeDtypeStruct((N, OUT), f32),
            jax.ShapeDtypeStruct((N, OUT), f32),
            jax.ShapeDtypeStruct((N, OUT), f32),
        ],
    )(x, s1[0:N], s1[N:2 * N], u1[0:N], u1[N:2 * N], cin2, cout2,
      W_self1, wi1, wo1, b1, wcat)

    s2sum, u2sum = _segsum_nocnt(p.reshape(2 * N, H), q.reshape(2 * N, H),
                                 src_g, src_g1, dst_g, dst_g1, src_s, dst_s)

    b2 = (b_self2 + (1.0 - ALPHA) * b_in2 + ALPHA * b_out2).reshape(1, OUT)
    out = pl.pallas_call(
        _tc3_body,
        grid=(N // M1,),
        in_specs=[
            _row_spec(OUT), _row_spec(H), _row_spec(H), _row_spec(H),
            _row_spec(H), _row_spec(1), _row_spec(1), _full_spec(1, OUT),
        ],
        out_specs=_row_spec(OUT),
        out_shape=jax.ShapeDtypeStruct((N, OUT), f32),
    )(s2, s2sum[0:N], s2sum[N:2 * N], u2sum[0:N], u2sum[N:2 * N],
      cin2, cout2, b2)
    return out
